# Initial kernel scaffold; baseline (speedup 1.0000x reference)
import jax
import jax.numpy as jnp
from jax.experimental import pallas as pl


def kernel(input_u, ua_embeddings, ia_embeddings):
    raise NotImplementedError("write your pallas kernel here")


# speedup vs baseline: 1.6512x; 1.6512x over previous
import functools

import jax
import jax.numpy as jnp
from jax import lax
from jax.experimental import pallas as pl
from jax.experimental.pallas import tpu as pltpu
from jax.experimental.pallas import tpu_sc as plsc

_C = 4
_D = 16
_NG = 21


def _make_sc_gather(n_ua_rows, n_ia_rows, batch):
    info = plsc.get_sparse_core_info()
    nw = info.num_cores * info.num_subcores
    bpw = batch // nw
    mesh = plsc.VectorSubcoreMesh(core_axis_name="c", subcore_axis_name="s")

    @functools.partial(
        pl.kernel,
        mesh=mesh,
        compiler_params=pltpu.CompilerParams(use_tc_tiling_on_sc=False),
        out_type=jax.ShapeDtypeStruct((_NG, batch, _D), jnp.float32),
        scratch_types=[
            pltpu.VMEM((bpw,), jnp.int32),
            pltpu.VMEM((bpw, _D), jnp.float32),
            pltpu.SemaphoreType.DMA,
        ],
    )
    def sc_gather(ua_hbm, ia_hbm, idx_hbm, out_hbm, idx_v, rows_v, sem):
        wid = lax.axis_index("s") * info.num_cores + lax.axis_index("c")
        base = wid * bpw
        for g in range(_NG):
            tab = ua_hbm if g < 7 else ia_hbm
            pltpu.sync_copy(idx_hbm.at[pl.ds(g * batch + base, bpw)], idx_v)
            pltpu.async_copy(tab.at[idx_v], rows_v, sem).wait()
            pltpu.sync_copy(rows_v, out_hbm.at[g, pl.ds(base, bpw), :])

    return sc_gather


def _tc_loss_body(g_ref, out_ref, acc_ref):
    step = pl.program_id(0)
    nsteps = pl.num_programs(0)

    @pl.when(step == 0)
    def _init():
        acc_ref[0, 0] = jnp.float32(0.0)

    lane = lax.broadcasted_iota(jnp.int32, (128, 8), 0)
    seg = lax.broadcasted_iota(jnp.int32, (128, 8), 1)
    S = (lane // _D == seg).astype(jnp.float32)

    def dot16(a, b):
        return jnp.dot(a * b, S, preferred_element_type=jnp.float32)

    sig = jax.nn.sigmoid
    rows = [g_ref[i] for i in range(_NG)]
    u = rows[0:3]
    u3 = rows[3:7]
    p = rows[7:10]
    n = rows[10:13]
    p3 = rows[13:17]
    n3 = rows[17:21]

    total = jnp.float32(0.0)
    for i in range(3):
        ps = sig(dot16(u[i], p[i]))
        ns = sig(dot16(u[i], n[i]))
        total += 0.01 * jnp.sum(-jax.nn.log_sigmoid(ps - ns))
        total += 0.5 * (jnp.sum(u[i] * u[i]) + jnp.sum(p[i] * p[i])
                        + jnp.sum(n[i] * n[i]))

    ue = u3[3]
    g2p = sig(dot16(ue, p3[3]))
    g2n = sig(dot16(ue, n3[3]))
    g1p = jnp.zeros_like(g2p)
    g1n = jnp.zeros_like(g2n)
    for k in range(3):
        sim = sig(dot16(ue, u3[k]))
        g1p += sim * sig(dot16(u3[k], p3[k]))
        g1n += sim * sig(dot16(u3[k], n3[k]))
    ps = (g1p / 3.0) * g2p
    ns = (g1n / 3.0) * g2n
    total += jnp.sum(-jax.nn.log_sigmoid(ps - ns))
    total += 0.5 * (jnp.sum(ue * ue) + jnp.sum(p3[3] * p3[3])
                    + jnp.sum(n3[3] * n3[3]))

    acc_ref[0, 0] += total

    @pl.when(step == nsteps - 1)
    def _fin():
        out_ref[0, 0] = acc_ref[0, 0]


def _tc_loss(gathered, batch):
    packed = gathered.reshape(_NG, batch // 8, 128)
    rows = batch // 8
    rblk = 256
    grid = rows // rblk
    out = pl.pallas_call(
        _tc_loss_body,
        grid=(grid,),
        in_specs=[pl.BlockSpec((_NG, rblk, 128), lambda b: (0, b, 0))],
        out_specs=pl.BlockSpec(memory_space=pltpu.SMEM),
        out_shape=jax.ShapeDtypeStruct((1, 1), jnp.float32),
        scratch_shapes=[pltpu.SMEM((1, 1), jnp.float32)],
    )(packed)
    return out[0, 0] / (batch * 5.0)


def kernel(input_u, ua_embeddings, ia_embeddings):
    n_users, c, d = ua_embeddings.shape
    n_items = ia_embeddings.shape[0]
    batch = input_u.shape[0]

    users, poss, negs = [], [], []
    for i in range(_C):
        key = jax.random.fold_in(jax.random.key(42), i)
        ku, kp, kn = jax.random.split(key, 3)
        users.append(jax.random.randint(ku, (batch,), 0, n_users))
        poss.append(jax.random.randint(kp, (batch,), 0, n_items))
        negs.append(jax.random.randint(kn, (batch,), 0, n_items))

    idx_list = []
    for i in range(3):
        idx_list.append(users[i] * _C + i)
    for k in range(_C):
        idx_list.append(users[3] * _C + k)
    for i in range(3):
        idx_list.append(poss[i] * _C + i)
    for i in range(3):
        idx_list.append(negs[i] * _C + i)
    for k in range(_C):
        idx_list.append(poss[3] * _C + k)
    for k in range(_C):
        idx_list.append(negs[3] * _C + k)
    idx_all = jnp.stack(idx_list).astype(jnp.int32).reshape(-1)

    ua_flat = ua_embeddings.reshape(n_users * c, d)
    ia_flat = ia_embeddings.reshape(n_items * c, d)

    sc_gather = _make_sc_gather(n_users * c, n_items * c, batch)
    gathered = sc_gather(ua_flat, ia_flat, idx_all)
    return _tc_loss(gathered, batch)
